# speedup vs baseline: 1.9702x; 1.9702x over previous
import jax
import jax.numpy as jnp
from jax import lax
from jax.experimental import pallas as pl
from jax.experimental.pallas import tpu as pltpu
from jax.experimental.pallas import tpu_sc as plsc

T_MAX = 1000
COND_DIM = 128
BATCH = 16384

NC = 2
NS = 16
LANES = 16
NW = NC * NS
B_PER_W = BATCH // NW
CHUNK = 128
N_CHUNKS = B_PER_W // CHUNK


def _emb_lookup_body(t_hbm, table_hbm, out_hbm, idx_flat, idx2, rows, sem):
    wid = lax.axis_index("s") * NC + lax.axis_index("c")
    base = wid * B_PER_W

    pltpu.sync_copy(t_hbm.at[pl.ds(base, B_PER_W)], idx_flat)

    for i in range(B_PER_W // LANES):
        v = idx_flat[pl.ds(i * LANES, LANES)]
        v = jnp.where(v == 0, T_MAX - 1, v - 1)
        idx2[i // (CHUNK // LANES), pl.ds((i % (CHUNK // LANES)) * LANES, LANES)] = v

    copies = []
    for j in range(N_CHUNKS):
        copies.append(
            pltpu.async_copy(
                table_hbm.at[idx2.at[j]], rows.at[pl.ds(j * CHUNK, CHUNK)], sem
            )
        )
    for c in copies:
        c.wait()

    pltpu.sync_copy(rows, out_hbm.at[pl.ds(base, B_PER_W)])


@jax.jit
def kernel(t, time_emb):
    mesh = plsc.VectorSubcoreMesh(
        core_axis_name="c", subcore_axis_name="s", num_cores=NC, num_subcores=NS
    )
    run = pl.kernel(
        _emb_lookup_body,
        out_type=jax.ShapeDtypeStruct((BATCH, COND_DIM), jnp.float32),
        mesh=mesh,
        scratch_types=[
            pltpu.VMEM((B_PER_W,), jnp.int32),
            pltpu.VMEM((N_CHUNKS, CHUNK), jnp.int32),
            pltpu.VMEM((B_PER_W, COND_DIM), jnp.float32),
            pltpu.SemaphoreType.DMA,
        ],
    )
    return run(t, time_emb)
